# Initial kernel scaffold; baseline (speedup 1.0000x reference)
import jax
import jax.numpy as jnp
from jax.experimental import pallas as pl


def kernel(x, scales):
    raise NotImplementedError("write your pallas kernel here")


# speedup vs baseline: 1.5100x; 1.5100x over previous
import jax
import jax.numpy as jnp
from jax.experimental import pallas as pl

NUM_POSITIONS = 8
CH = 16
ROW = NUM_POSITIONS * CH

BLK = 2048


def _body(wa_ref, wb_ref, x_ref, o_ref):
    blk = x_ref[...]
    wa = wa_ref[...]
    wb = wb_ref[...]
    a = jnp.sum(blk * wa, axis=1, keepdims=True)
    b = jnp.sum(blk * wb, axis=1, keepdims=True)
    lane = jax.lax.broadcasted_iota(jnp.int32, (1, ROW), 1) % CH
    out = jnp.where(lane == 12, a, blk)
    out = jnp.where(lane == 13, b, out)
    o_ref[...] = out


def kernel(x, scales):
    B = x.shape[0]
    x2 = x.reshape(B, ROW)
    lane = jax.lax.iota(jnp.int32, ROW)
    pos = lane // CH
    ch = lane % CH
    sc = scales[pos]
    wa = jnp.where(ch == 0, sc, 0.0).reshape(1, ROW)
    wb = jnp.where(ch == 1, sc, 0.0).reshape(1, ROW)
    grid = (B // BLK,)
    y2 = pl.pallas_call(
        _body,
        grid=grid,
        in_specs=[
            pl.BlockSpec((1, ROW), lambda i: (0, 0)),
            pl.BlockSpec((1, ROW), lambda i: (0, 0)),
            pl.BlockSpec((BLK, ROW), lambda i: (i, 0)),
        ],
        out_specs=pl.BlockSpec((BLK, ROW), lambda i: (i, 0)),
        out_shape=jax.ShapeDtypeStruct((B, ROW), x.dtype),
    )(wa, wb, x2)
    return y2.reshape(B, NUM_POSITIONS, CH)


# speedup vs baseline: 1.7641x; 1.1683x over previous
import jax
import jax.numpy as jnp
from jax.experimental import pallas as pl

NUM_POSITIONS = 8
CH = 16
ROW = NUM_POSITIONS * CH

BLK = 8192


def _body(wa_ref, wb_ref, x_ref, o_ref):
    blk = x_ref[...]
    wa = wa_ref[...]
    wb = wb_ref[...]
    a = jnp.sum(blk * wa, axis=1, keepdims=True)
    b = jnp.sum(blk * wb, axis=1, keepdims=True)
    lane = jax.lax.broadcasted_iota(jnp.int32, (1, ROW), 1) % CH
    out = jnp.where(lane == 12, a, blk)
    out = jnp.where(lane == 13, b, out)
    o_ref[...] = out


def kernel(x, scales):
    B = x.shape[0]
    x2 = x.reshape(B, ROW)
    lane = jax.lax.iota(jnp.int32, ROW)
    pos = lane // CH
    ch = lane % CH
    sc = scales[pos]
    wa = jnp.where(ch == 0, sc, 0.0).reshape(1, ROW)
    wb = jnp.where(ch == 1, sc, 0.0).reshape(1, ROW)
    grid = (B // BLK,)
    y2 = pl.pallas_call(
        _body,
        grid=grid,
        in_specs=[
            pl.BlockSpec((1, ROW), lambda i: (0, 0)),
            pl.BlockSpec((1, ROW), lambda i: (0, 0)),
            pl.BlockSpec((BLK, ROW), lambda i: (i, 0)),
        ],
        out_specs=pl.BlockSpec((BLK, ROW), lambda i: (i, 0)),
        out_shape=jax.ShapeDtypeStruct((B, ROW), x.dtype),
    )(wa, wb, x2)
    return y2.reshape(B, NUM_POSITIONS, CH)


# speedup vs baseline: 1.8040x; 1.0226x over previous
import jax
import jax.numpy as jnp
from jax.experimental import pallas as pl

NUM_POSITIONS = 8
CH = 16
ROW = NUM_POSITIONS * CH

BLK = 16384


def _body(wa_ref, wb_ref, x_ref, o_ref):
    blk = x_ref[...]
    wa = wa_ref[...]
    wb = wb_ref[...]
    a = jnp.sum(blk * wa, axis=1, keepdims=True)
    b = jnp.sum(blk * wb, axis=1, keepdims=True)
    lane = jax.lax.broadcasted_iota(jnp.int32, (1, ROW), 1) % CH
    out = jnp.where(lane == 12, a, blk)
    out = jnp.where(lane == 13, b, out)
    o_ref[...] = out


def kernel(x, scales):
    B = x.shape[0]
    x2 = x.reshape(B, ROW)
    lane = jax.lax.iota(jnp.int32, ROW)
    pos = lane // CH
    ch = lane % CH
    sc = scales[pos]
    wa = jnp.where(ch == 0, sc, 0.0).reshape(1, ROW)
    wb = jnp.where(ch == 1, sc, 0.0).reshape(1, ROW)
    grid = (B // BLK,)
    y2 = pl.pallas_call(
        _body,
        grid=grid,
        in_specs=[
            pl.BlockSpec((1, ROW), lambda i: (0, 0)),
            pl.BlockSpec((1, ROW), lambda i: (0, 0)),
            pl.BlockSpec((BLK, ROW), lambda i: (i, 0)),
        ],
        out_specs=pl.BlockSpec((BLK, ROW), lambda i: (i, 0)),
        out_shape=jax.ShapeDtypeStruct((B, ROW), x.dtype),
    )(wa, wb, x2)
    return y2.reshape(B, NUM_POSITIONS, CH)
